# baseline (speedup 1.0000x reference)
import jax
import jax.numpy as jnp
from jax import lax
from jax.experimental import pallas as pl
from jax.experimental.pallas import tpu as pltpu
from jax.experimental.pallas import tpu_sc as plsc

DIM = 64
SCALE = 8.0

NC = 2
NS = 16
NW = NC * NS
SEQ = 200
BATCH = 4096
NTI = 800
TPW = NTI // NW
TSUB = 2
NSTEP = TPW * (8 // TSUB)


def _body(ids_hbm, tab_hbm, out_hbm, idx_v, rows_v, tile_v, gsem, ssem):
  cid = lax.axis_index("c")
  sid = lax.axis_index("s")
  wid = sid * NC + cid

  pltpu.sync_copy(ids_hbm.at[pl.ds(wid * TPW, TPW)], idx_v)

  def fire_gather(step, b):
    k = step // 4
    q = lax.rem(step, 4)
    for j in range(TSUB):
      pltpu.async_copy(
          tab_hbm.at[idx_v.at[k, q * TSUB + j]],
          rows_v.at[b, j],
          gsem.at[b],
      )

  def wait_gather(b):
    for j in range(TSUB):
      pltpu.make_async_copy(
          tab_hbm.at[idx_v.at[0, 0]], rows_v.at[b, j], gsem.at[b]
      ).wait()

  def fire_store(step, b):
    k = step // 4
    q = lax.rem(step, 4)
    ft = wid * TPW + k
    tr = ft // 32
    tc = lax.rem(ft, 32)
    pltpu.async_copy(
        tile_v.at[b],
        out_hbm.at[pl.ds(8 * tr + TSUB * q, TSUB), pl.ds(0, 8),
                   pl.ds(tc, 1), pl.ds(0, 8), pl.ds(0, 128)],
        ssem.at[b],
    )

  def wait_store(b):
    pltpu.make_async_copy(
        tile_v.at[b],
        out_hbm.at[pl.ds(0, TSUB), pl.ds(0, 8), pl.ds(0, 1),
                   pl.ds(0, 8), pl.ds(0, 128)],
        ssem.at[b],
    ).wait()

  iota16 = lax.iota(jnp.int32, 16)
  zeros16 = jnp.zeros((16,), jnp.int32)

  def xpose(b):
    bvec = jnp.full((16,), b, jnp.int32)

    @plsc.parallel_loop(0, TSUB * 512, unroll=4)
    def _(i):
      ti = i >> 9
      rem = i & 511
      f = rem >> 3
      b0 = (rem & 7) << 4
      r = f >> 3
      fr = f & 7
      v = plsc.load_gather(
          rows_v,
          [bvec, zeros16 + ti, b0 + iota16, zeros16 + f],
      )
      tile_v[b, ti, r, 0, fr, pl.ds(b0 * 1, 16)] = v * SCALE

  fire_gather(0, 0)

  @pl.loop(0, NSTEP, step=2)
  def _(s):
    for b in range(2):
      ss = s + b

      @pl.when(ss + 1 < NSTEP)
      def _():
        @pl.when(ss >= 1)
        def _():
          wait_store(1 - b)
        fire_gather(ss + 1, 1 - b)

      wait_gather(b)
      xpose(b)
      fire_store(ss, b)

  wait_store(0)
  wait_store(1)


@jax.jit
def _embed(ids_in, tab2):
  mesh = plsc.VectorSubcoreMesh(core_axis_name="c", subcore_axis_name="s")
  run = pl.kernel(
      _body,
      out_type=jax.ShapeDtypeStruct((SEQ, 8, 32, 8, 128), jnp.float32),
      mesh=mesh,
      scratch_types=[
          pltpu.VMEM((TPW, 8, 128), jnp.int32),
          pltpu.VMEM((2, TSUB, 128, 128), jnp.float32),
          pltpu.VMEM((2, TSUB, 8, 1, 8, 128), jnp.float32),
          pltpu.SemaphoreType.DMA((2,)),
          pltpu.SemaphoreType.DMA((2,)),
      ],
      compiler_params=pltpu.CompilerParams(
          use_tc_tiling_on_sc=True, needs_layout_passes=False),
  )
  return run(ids_in, tab2)


def kernel(token_ids, tok_embedding):
  ids_in = (jnp.transpose(token_ids).reshape(25, 8, 32, 128)
            .transpose(0, 2, 1, 3).reshape(NTI, 8, 128)
            .astype(jnp.int32))
  tab2 = jnp.pad(tok_embedding, ((0, 0), (0, 128 - DIM)))
  o = _embed(ids_in, tab2)
  return o.transpose(2, 4, 0, 1, 3).reshape(BATCH, SEQ, DIM)
